# baseline (speedup 1.0000x reference)
import functools

import jax
import jax.numpy as jnp
from jax import lax
from jax.experimental import pallas as pl
from jax.experimental.pallas import tpu as pltpu
from jax.experimental.pallas import tpu_sc as plsc

B = 16384
D = 64
K = 20
V = 1000000
NW = 32
BPW = B // NW
CH = 16
NCHUNK = BPW // CH
NGI = 5
GR = CH * K // NGI

TR_C = 8192


def _tc_concat_t(wht_ref, wot_ref, out_ref):
    ri = lax.broadcasted_iota(jnp.int32, (D, D), 0)
    ci = lax.broadcasted_iota(jnp.int32, (D, D), 1)
    eye = jnp.where(ri == ci, 1.0, 0.0).astype(jnp.float32)
    dn = (((0,), (0,)), ((), ()))
    out_ref[:, :D] = lax.dot_general(wht_ref[...], eye, dn,
                                     preferred_element_type=jnp.float32)
    out_ref[:, D:] = wot_ref[...].T


def _build_table(w_hidden_t, w_output_t):
    return pl.pallas_call(
        _tc_concat_t,
        grid=(pl.cdiv(V, TR_C),),
        in_specs=[
            pl.BlockSpec((D, TR_C), lambda i: (0, i)),
            pl.BlockSpec((D, TR_C), lambda i: (0, i)),
        ],
        out_specs=pl.BlockSpec((TR_C, 2 * D), lambda i: (i, 0)),
        out_shape=jax.ShapeDtypeStruct((V, 2 * D), jnp.float32),
    )(w_hidden_t, w_output_t)


def _sc_scores(tgt, ctx, negf, w_both):
    mesh = plsc.VectorSubcoreMesh(core_axis_name="c", subcore_axis_name="s")

    buf_set = (
        pltpu.VMEM((CH,), jnp.int32),
        pltpu.VMEM((CH,), jnp.int32),
        pltpu.VMEM((CH * K,), jnp.int32),
        pltpu.VMEM((CH, 128), jnp.float32),
        pltpu.VMEM((CH, 128), jnp.float32),
        pltpu.VMEM((CH * K, 128), jnp.float32),
        pltpu.SemaphoreType.DMA,
    )

    @functools.partial(
        pl.kernel,
        out_type=(
            jax.ShapeDtypeStruct((B * 16,), jnp.float32),
            jax.ShapeDtypeStruct((B * 16,), jnp.float32),
        ),
        mesh=mesh,
        scratch_types=buf_set + buf_set + (
            pltpu.VMEM((CH * 16,), jnp.float32),
            pltpu.VMEM((CH * 16,), jnp.float32),
        ),
    )
    def sc_k(tgt_hbm, ctx_hbm, negf_hbm, wb_hbm, pos_out, neg_out,
             ta, ca, na, rha, roa, nba, sema,
             tb, cb, nb, rhb, rob, nbb, semb,
             sc_p, sc_n):
        wid = lax.axis_index("s") * 2 + lax.axis_index("c")

        def issue(c, idx_t, idx_c, idx_n, rows_h, rows_o, negbuf, sem):
            base = wid * BPW + c * CH
            pltpu.sync_copy(tgt_hbm.at[pl.ds(base, CH)], idx_t)
            pltpu.sync_copy(ctx_hbm.at[pl.ds(base, CH)], idx_c)
            pltpu.sync_copy(negf_hbm.at[pl.ds(base * K, CH * K)], idx_n)
            pltpu.async_copy(wb_hbm.at[idx_t], rows_h, sem)
            pltpu.async_copy(wb_hbm.at[idx_c], rows_o, sem)
            for i in range(NGI):
                pltpu.async_copy(
                    wb_hbm.at[idx_n.at[pl.ds(i * GR, GR)]],
                    negbuf.at[pl.ds(i * GR, GR)], sem)

        def drain(idx_t, idx_c, idx_n, rows_h, rows_o, negbuf, sem):
            pltpu.make_async_copy(wb_hbm.at[idx_t], rows_h, sem).wait()
            pltpu.make_async_copy(wb_hbm.at[idx_c], rows_o, sem).wait()
            for i in range(NGI):
                pltpu.make_async_copy(
                    wb_hbm.at[idx_n.at[pl.ds(i * GR, GR)]],
                    negbuf.at[pl.ds(i * GR, GR)], sem).wait()

        def compute(c, rows_h, rows_o, negbuf):
            base = wid * BPW + c * CH

            def b_body(b, carry2):
                h0 = rows_h[b, pl.ds(0, 16)]
                h1 = rows_h[b, pl.ds(16, 16)]
                h2 = rows_h[b, pl.ds(32, 16)]
                h3 = rows_h[b, pl.ds(48, 16)]
                accp = (h0 * rows_o[b, pl.ds(64, 16)]
                        + h1 * rows_o[b, pl.ds(80, 16)]
                        + h2 * rows_o[b, pl.ds(96, 16)]
                        + h3 * rows_o[b, pl.ds(112, 16)])
                accn = jnp.zeros((16,), jnp.float32)
                for j in range(K):
                    r = b * K + j
                    accn = accn + (h0 * negbuf[r, pl.ds(64, 16)]
                                   + h1 * negbuf[r, pl.ds(80, 16)]
                                   + h2 * negbuf[r, pl.ds(96, 16)]
                                   + h3 * negbuf[r, pl.ds(112, 16)])
                sc_p[pl.ds(b * 16, 16)] = accp
                sc_n[pl.ds(b * 16, 16)] = accn
                return carry2

            lax.fori_loop(0, CH, b_body, 0)
            pltpu.sync_copy(sc_p, pos_out.at[pl.ds(base * 16, CH * 16)])
            pltpu.sync_copy(sc_n, neg_out.at[pl.ds(base * 16, CH * 16)])

        bufs_a = (ta, ca, na, rha, roa, nba, sema)
        bufs_b = (tb, cb, nb, rhb, rob, nbb, semb)
        issue(0, *bufs_a)

        def pair_body(i2, carry):
            c0 = 2 * i2
            issue(c0 + 1, *bufs_b)
            drain(*bufs_a[:6], bufs_a[6])
            compute(c0, rha, roa, nba)

            @pl.when(c0 + 2 < NCHUNK)
            def _():
                issue(c0 + 2, *bufs_a)

            drain(*bufs_b[:6], bufs_b[6])
            compute(c0 + 1, rhb, rob, nbb)
            return carry

        lax.fori_loop(0, NCHUNK // 2, pair_body, 0)

    return sc_k(tgt, ctx, negf, w_both)


def _tc_loss(pos_ref, neg_ref, out_ref):
    ri = lax.broadcasted_iota(jnp.int32, (128, 8), 0)
    ci = lax.broadcasted_iota(jnp.int32, (128, 8), 1)
    m = jnp.where(ri // 16 == ci, 1.0, 0.0).astype(jnp.float32)
    sp = jnp.dot(pos_ref[...], m, preferred_element_type=jnp.float32)
    sn = jnp.dot(neg_ref[...], m, preferred_element_type=jnp.float32)
    s = jnp.sum(jax.nn.log_sigmoid(sp)) + jnp.sum(jax.nn.log_sigmoid(-sn))
    out_ref[0, 0] = -s


def kernel(targets, contexts, neg_samples, W_hidden, W_output):
    tgt = targets.astype(jnp.int32)
    ctx = contexts.astype(jnp.int32)
    negf = neg_samples.astype(jnp.int32).reshape(B * K)
    w_both = _build_table(W_hidden.T, W_output.T)
    pos, neg = _sc_scores(tgt, ctx, negf, w_both)
    out = pl.pallas_call(
        _tc_loss,
        out_shape=jax.ShapeDtypeStruct((1, 1), jnp.float32),
        out_specs=pl.BlockSpec(memory_space=pltpu.SMEM),
    )(pos.reshape(B // 8, 128), neg.reshape(B // 8, 128))
    return out[0, 0]


# speedup vs baseline: 1.0580x; 1.0580x over previous
import functools

import jax
import jax.numpy as jnp
from jax import lax
from jax.experimental import pallas as pl
from jax.experimental.pallas import tpu as pltpu
from jax.experimental.pallas import tpu_sc as plsc

B = 16384
D = 64
K = 20
V = 1000000
NW = 32
BPW = B // NW
CH = 16
NCHUNK = BPW // CH
NGI = 5
GR = CH * K // NGI

TR_C = 16384


def _tc_concat_t(wht_ref, wot_ref, out_ref):
    ri = lax.broadcasted_iota(jnp.int32, (D, D), 0)
    ci = lax.broadcasted_iota(jnp.int32, (D, D), 1)
    eye = jnp.where(ri == ci, 1.0, 0.0).astype(jnp.float32)
    dn = (((0,), (0,)), ((), ()))
    out_ref[:, :D] = lax.dot_general(wht_ref[...], eye, dn,
                                     preferred_element_type=jnp.float32)
    out_ref[:, D:] = wot_ref[...].T


def _build_table(w_hidden_t, w_output_t):
    return pl.pallas_call(
        _tc_concat_t,
        grid=(pl.cdiv(V, TR_C),),
        in_specs=[
            pl.BlockSpec((D, TR_C), lambda i: (0, i)),
            pl.BlockSpec((D, TR_C), lambda i: (0, i)),
        ],
        out_specs=pl.BlockSpec((TR_C, 2 * D), lambda i: (i, 0)),
        out_shape=jax.ShapeDtypeStruct((V, 2 * D), jnp.float32),
    )(w_hidden_t, w_output_t)


def _sc_scores(tgt, ctx, negf, w_both):
    mesh = plsc.VectorSubcoreMesh(core_axis_name="c", subcore_axis_name="s")

    buf_set = (
        pltpu.VMEM((CH,), jnp.int32),
        pltpu.VMEM((CH,), jnp.int32),
        pltpu.VMEM((CH * K,), jnp.int32),
        pltpu.VMEM((CH, 128), jnp.float32),
        pltpu.VMEM((CH, 128), jnp.float32),
        pltpu.VMEM((CH * K, 128), jnp.float32),
        pltpu.SemaphoreType.DMA,
    )

    @functools.partial(
        pl.kernel,
        out_type=(
            jax.ShapeDtypeStruct((B * 16,), jnp.float32),
            jax.ShapeDtypeStruct((B * 16,), jnp.float32),
        ),
        mesh=mesh,
        scratch_types=buf_set + buf_set + (
            pltpu.VMEM((CH * 16,), jnp.float32),
            pltpu.VMEM((CH * 16,), jnp.float32),
        ),
    )
    def sc_k(tgt_hbm, ctx_hbm, negf_hbm, wb_hbm, pos_out, neg_out,
             ta, ca, na, rha, roa, nba, sema,
             tb, cb, nb, rhb, rob, nbb, semb,
             sc_p, sc_n):
        wid = lax.axis_index("s") * 2 + lax.axis_index("c")

        def issue(c, idx_t, idx_c, idx_n, rows_h, rows_o, negbuf, sem):
            base = wid * BPW + c * CH
            pltpu.sync_copy(tgt_hbm.at[pl.ds(base, CH)], idx_t)
            pltpu.sync_copy(ctx_hbm.at[pl.ds(base, CH)], idx_c)
            pltpu.sync_copy(negf_hbm.at[pl.ds(base * K, CH * K)], idx_n)
            pltpu.async_copy(wb_hbm.at[idx_t], rows_h, sem)
            pltpu.async_copy(wb_hbm.at[idx_c], rows_o, sem)
            for i in range(NGI):
                pltpu.async_copy(
                    wb_hbm.at[idx_n.at[pl.ds(i * GR, GR)]],
                    negbuf.at[pl.ds(i * GR, GR)], sem)

        def drain(idx_t, idx_c, idx_n, rows_h, rows_o, negbuf, sem):
            pltpu.make_async_copy(wb_hbm.at[idx_t], rows_h, sem).wait()
            pltpu.make_async_copy(wb_hbm.at[idx_c], rows_o, sem).wait()
            for i in range(NGI):
                pltpu.make_async_copy(
                    wb_hbm.at[idx_n.at[pl.ds(i * GR, GR)]],
                    negbuf.at[pl.ds(i * GR, GR)], sem).wait()

        def compute(c, rows_h, rows_o, negbuf):
            base = wid * BPW + c * CH

            def b_body(b, carry2):
                h0 = rows_h[b, pl.ds(0, 16)]
                h1 = rows_h[b, pl.ds(16, 16)]
                h2 = rows_h[b, pl.ds(32, 16)]
                h3 = rows_h[b, pl.ds(48, 16)]
                accp = (h0 * rows_o[b, pl.ds(64, 16)]
                        + h1 * rows_o[b, pl.ds(80, 16)]
                        + h2 * rows_o[b, pl.ds(96, 16)]
                        + h3 * rows_o[b, pl.ds(112, 16)])
                accn = jnp.zeros((16,), jnp.float32)
                for j in range(K):
                    r = b * K + j
                    accn = accn + (h0 * negbuf[r, pl.ds(64, 16)]
                                   + h1 * negbuf[r, pl.ds(80, 16)]
                                   + h2 * negbuf[r, pl.ds(96, 16)]
                                   + h3 * negbuf[r, pl.ds(112, 16)])
                sc_p[pl.ds(b * 16, 16)] = accp
                sc_n[pl.ds(b * 16, 16)] = accn
                return carry2

            lax.fori_loop(0, CH, b_body, 0)
            pltpu.sync_copy(sc_p, pos_out.at[pl.ds(base * 16, CH * 16)])
            pltpu.sync_copy(sc_n, neg_out.at[pl.ds(base * 16, CH * 16)])

        bufs_a = (ta, ca, na, rha, roa, nba, sema)
        bufs_b = (tb, cb, nb, rhb, rob, nbb, semb)
        issue(0, *bufs_a)

        def pair_body(i2, carry):
            c0 = 2 * i2
            issue(c0 + 1, *bufs_b)
            drain(*bufs_a[:6], bufs_a[6])
            compute(c0, rha, roa, nba)

            @pl.when(c0 + 2 < NCHUNK)
            def _():
                issue(c0 + 2, *bufs_a)

            drain(*bufs_b[:6], bufs_b[6])
            compute(c0 + 1, rhb, rob, nbb)
            return carry

        lax.fori_loop(0, NCHUNK // 2, pair_body, 0)

    return sc_k(tgt, ctx, negf, w_both)


def _tc_loss(pos_ref, neg_ref, out_ref):
    ri = lax.broadcasted_iota(jnp.int32, (128, 8), 0)
    ci = lax.broadcasted_iota(jnp.int32, (128, 8), 1)
    m = jnp.where(ri // 16 == ci, 1.0, 0.0).astype(jnp.float32)
    sp = jnp.dot(pos_ref[...], m, preferred_element_type=jnp.float32)
    sn = jnp.dot(neg_ref[...], m, preferred_element_type=jnp.float32)
    s = jnp.sum(jax.nn.log_sigmoid(sp)) + jnp.sum(jax.nn.log_sigmoid(-sn))
    out_ref[0, 0] = -s


def kernel(targets, contexts, neg_samples, W_hidden, W_output):
    tgt = targets.astype(jnp.int32)
    ctx = contexts.astype(jnp.int32)
    negf = neg_samples.astype(jnp.int32).reshape(B * K)
    w_both = _build_table(W_hidden.T, W_output.T)
    pos, neg = _sc_scores(tgt, ctx, negf, w_both)
    out = pl.pallas_call(
        _tc_loss,
        out_shape=jax.ShapeDtypeStruct((1, 1), jnp.float32),
        out_specs=pl.BlockSpec(memory_space=pltpu.SMEM),
    )(pos.reshape(B // 8, 128), neg.reshape(B // 8, 128))
    return out[0, 0]
